# Initial kernel scaffold; baseline (speedup 1.0000x reference)
import jax
import jax.numpy as jnp
from jax.experimental import pallas as pl


def kernel(labels, class_means, class_stds, noise):
    raise NotImplementedError("write your pallas kernel here")


# speedup vs baseline: 11.0945x; 11.0945x over previous
import functools

import jax
import jax.numpy as jnp
from jax import lax
from jax.experimental import pallas as pl
from jax.experimental.pallas import tpu as pltpu
from jax.experimental.pallas import tpu_sc as plsc

NUM_CLASSES = 100000
C, H, W = 4, 8, 8
D = C * H * W
B = 16384
NC, NS = 2, 16
NW = NC * NS
BPW = B // NW
CH = 128
NCHUNK = BPW // CH
LANES = 16


def _sc_body(labels_hbm, means_hbm, stds_hbm, noise_hbm, out_hbm,
             idx_v, means_v, stds_v, noise_v, sem_m, sem_s, sem_n):
    cid = lax.axis_index("c")
    sid = lax.axis_index("s")
    wid = sid * NC + cid
    base = wid * BPW

    pltpu.sync_copy(labels_hbm.at[wid], idx_v)

    def chunk(c, carry):
        row0 = base + c * CH
        cm = pltpu.async_copy(means_hbm.at[idx_v.at[c]], means_v, sem_m)
        cs = pltpu.async_copy(stds_hbm.at[idx_v.at[c]], stds_v, sem_s)
        cn = pltpu.async_copy(noise_hbm.at[pl.ds(row0, CH)], noise_v, sem_n)
        cm.wait()
        cs.wait()
        cn.wait()

        def fma_row(r, carry2):
            for j in range(D // LANES):
                sl = (r, pl.ds(j * LANES, LANES))
                means_v[sl] = means_v[sl] + stds_v[sl] * noise_v[sl]
            return carry2

        lax.fori_loop(0, CH, fma_row, 0)
        pltpu.sync_copy(means_v, out_hbm.at[pl.ds(row0, CH)])
        return carry

    lax.fori_loop(0, NCHUNK, chunk, 0)


@functools.partial(jax.jit)
def _sc_call(labels_r, means2, stds2, noise2):
    f = functools.partial(
        pl.kernel,
        out_type=jax.ShapeDtypeStruct((B, D), jnp.float32),
        mesh=plsc.VectorSubcoreMesh(
            core_axis_name="c", subcore_axis_name="s",
            num_cores=NC, num_subcores=NS),
        scratch_types=[
            pltpu.VMEM((NCHUNK, CH), jnp.int32),
            pltpu.VMEM((CH, D), jnp.float32),
            pltpu.VMEM((CH, D), jnp.float32),
            pltpu.VMEM((CH, D), jnp.float32),
            pltpu.SemaphoreType.DMA,
            pltpu.SemaphoreType.DMA,
            pltpu.SemaphoreType.DMA,
        ],
    )(_sc_body)
    return f(labels_r, means2, stds2, noise2)


def kernel(labels, class_means, class_stds, noise):
    means2 = class_means.reshape(NUM_CLASSES, D)
    stds2 = class_stds.reshape(NUM_CLASSES, D)
    noise2 = noise.reshape(B, D)
    labels_r = labels.astype(jnp.int32).reshape(NW, NCHUNK, CH)
    out = _sc_call(labels_r, means2, stds2, noise2)
    return out.reshape(B, C, H, W)


# speedup vs baseline: 11.3750x; 1.0253x over previous
import functools

import jax
import jax.numpy as jnp
from jax import lax
from jax.experimental import pallas as pl
from jax.experimental.pallas import tpu as pltpu
from jax.experimental.pallas import tpu_sc as plsc

NUM_CLASSES = 100000
C, H, W = 4, 8, 8
D = C * H * W
B = 16384
NC, NS = 2, 16
NW = NC * NS
BPW = B // NW
CH = 32
NCHUNK = BPW // CH
LANES = 16


def _sc_body(labels_hbm, means_hbm, stds_hbm, noise_hbm, out_hbm,
             idx_v, means_v, stds_v, noise_v, out_v, sem_in, sem_out):
    cid = lax.axis_index("c")
    sid = lax.axis_index("s")
    wid = sid * NC + cid
    base = wid * BPW

    pltpu.sync_copy(labels_hbm.at[wid], idx_v)

    in_copies = [None] * NCHUNK
    out_copies = [None] * NCHUNK

    def issue(c):
        b = c & 1
        row0 = base + c * CH
        in_copies[c] = (
            pltpu.async_copy(means_hbm.at[idx_v.at[c]], means_v.at[b], sem_in[b]),
            pltpu.async_copy(stds_hbm.at[idx_v.at[c]], stds_v.at[b], sem_in[b]),
            pltpu.async_copy(noise_hbm.at[pl.ds(row0, CH)], noise_v.at[b], sem_in[b]),
        )

    issue(0)
    for c in range(NCHUNK):
        b = c & 1
        if c + 1 < NCHUNK:
            issue(c + 1)
        for h in in_copies[c]:
            h.wait()
        if c >= 2:
            out_copies[c - 2].wait()

        def fma_row(r, carry):
            for j in range(D // LANES):
                sl = (r, pl.ds(j * LANES, LANES))
                out_v[b, sl[0], sl[1]] = (
                    means_v[b, sl[0], sl[1]]
                    + stds_v[b, sl[0], sl[1]] * noise_v[b, sl[0], sl[1]])
            return carry

        lax.fori_loop(0, CH, fma_row, 0)
        row0 = base + c * CH
        out_copies[c] = pltpu.async_copy(
            out_v.at[b], out_hbm.at[pl.ds(row0, CH)], sem_out[b])

    out_copies[NCHUNK - 2].wait()
    out_copies[NCHUNK - 1].wait()


@functools.partial(jax.jit)
def _sc_call(labels_r, means2, stds2, noise2):
    f = functools.partial(
        pl.kernel,
        out_type=jax.ShapeDtypeStruct((B, D), jnp.float32),
        mesh=plsc.VectorSubcoreMesh(
            core_axis_name="c", subcore_axis_name="s",
            num_cores=NC, num_subcores=NS),
        scratch_types=[
            pltpu.VMEM((NCHUNK, CH), jnp.int32),
            pltpu.VMEM((2, CH, D), jnp.float32),
            pltpu.VMEM((2, CH, D), jnp.float32),
            pltpu.VMEM((2, CH, D), jnp.float32),
            pltpu.VMEM((2, CH, D), jnp.float32),
            (pltpu.SemaphoreType.DMA, pltpu.SemaphoreType.DMA),
            (pltpu.SemaphoreType.DMA, pltpu.SemaphoreType.DMA),
        ],
    )(_sc_body)
    return f(labels_r, means2, stds2, noise2)


def kernel(labels, class_means, class_stds, noise):
    means2 = class_means.reshape(NUM_CLASSES, D)
    stds2 = class_stds.reshape(NUM_CLASSES, D)
    noise2 = noise.reshape(B, D)
    labels_r = labels.astype(jnp.int32).reshape(NW, NCHUNK, CH)
    out = _sc_call(labels_r, means2, stds2, noise2)
    return out.reshape(B, C, H, W)
